# baseline (speedup 1.0000x reference)
import functools

import jax
import jax.numpy as jnp
from jax import lax
from jax.experimental import pallas as pl
from jax.experimental.pallas import tpu as pltpu
from jax.experimental.pallas import tpu_sc as plsc


@functools.lru_cache(maxsize=None)
def _make_sc_gather(B: int, E: int):
    info = plsc.get_sparse_core_info()
    nw = info.num_cores * info.num_subcores
    bpw = B // nw
    ch = 128 if bpw % 128 == 0 else bpw
    nch = bpw // ch
    mesh = plsc.VectorSubcoreMesh(core_axis_name="c", subcore_axis_name="s")

    @functools.partial(
        pl.kernel,
        mesh=mesh,
        compiler_params=pltpu.CompilerParams(use_tc_tiling_on_sc=False),
        out_type=(
            jax.ShapeDtypeStruct((B, E), jnp.float32),
            jax.ShapeDtypeStruct((B, E), jnp.float32),
        ),
        scratch_types=[
            pltpu.VMEM((nch, ch), jnp.int32),
            pltpu.VMEM((nch, ch), jnp.int32),
            pltpu.VMEM((bpw, E), jnp.float32),
            pltpu.VMEM((bpw, E), jnp.float32),
            pltpu.SemaphoreType.DMA,
            pltpu.SemaphoreType.DMA,
        ],
    )
    def sc_gather(uid_hbm, mid_hbm, utab_hbm, mtab_hbm, u_out, m_out,
                  uidx, midx, urows, mrows, usem, msem):
        wid = lax.axis_index("s") * info.num_cores + lax.axis_index("c")
        base = wid * bpw
        for j in range(nch):
            pltpu.sync_copy(uid_hbm.at[pl.ds(base + j * ch, ch)], uidx.at[j])
            pltpu.sync_copy(mid_hbm.at[pl.ds(base + j * ch, ch)], midx.at[j])
        copies = []
        for j in range(nch):
            copies.append(pltpu.async_copy(
                utab_hbm.at[uidx.at[j]], urows.at[pl.ds(j * ch, ch)], usem))
            copies.append(pltpu.async_copy(
                mtab_hbm.at[midx.at[j]], mrows.at[pl.ds(j * ch, ch)], msem))
        for c in copies:
            c.wait()
        pltpu.sync_copy(urows, u_out.at[pl.ds(base, bpw)])
        pltpu.sync_copy(mrows, m_out.at[pl.ds(base, bpw)])

    return sc_gather


@functools.lru_cache(maxsize=None)
def _make_mlp(B: int, E: int, G: int, NG: int, H1: int, H2: int, bt: int):
    prec = lax.Precision.HIGHEST

    def body(u_ref, m_ref, gen_ref, glen_ref, gt_ref, w1_ref, b1_ref,
             w2_ref, b2_ref, wfc_ref, bfc_ref, out_ref):
        f32 = jnp.float32
        glen = glen_ref[...]
        inv_len = 1.0 / jnp.maximum(glen, 1).astype(f32)
        iota = lax.broadcasted_iota(jnp.int32, (bt, NG), 1)
        gen = gen_ref[...]
        onehot = jnp.zeros((bt, NG), f32)
        for j in range(G):
            gj = gen[:, j:j + 1]
            wj = jnp.where(j < glen, inv_len, 0.0)
            onehot = onehot + jnp.where(gj == iota, wj, 0.0)
        gbag = jnp.dot(onehot, gt_ref[...],
                       preferred_element_type=f32, precision=prec)
        u = u_ref[...]
        mr = m_ref[...] + gbag
        w1 = w1_ref[...]
        h1 = (jnp.dot(u, w1[:E, :], preferred_element_type=f32, precision=prec)
              + jnp.dot(mr, w1[E:, :], preferred_element_type=f32, precision=prec)
              + b1_ref[...])
        h1 = jnp.maximum(h1, 0.0)
        h2 = jnp.dot(h1, w2_ref[...], preferred_element_type=f32,
                     precision=prec) + b2_ref[...]
        o = jnp.dot(h2, wfc_ref[...], preferred_element_type=f32,
                    precision=prec) + bfc_ref[...]
        out_ref[...] = o

    return pl.pallas_call(
        body,
        grid=(B // bt,),
        in_specs=[
            pl.BlockSpec((bt, E), lambda i: (i, 0)),
            pl.BlockSpec((bt, E), lambda i: (i, 0)),
            pl.BlockSpec((bt, G), lambda i: (i, 0)),
            pl.BlockSpec((bt, 1), lambda i: (i, 0)),
            pl.BlockSpec((NG, E), lambda i: (0, 0)),
            pl.BlockSpec((2 * E, H1), lambda i: (0, 0)),
            pl.BlockSpec((1, H1), lambda i: (0, 0)),
            pl.BlockSpec((H1, H2), lambda i: (0, 0)),
            pl.BlockSpec((1, H2), lambda i: (0, 0)),
            pl.BlockSpec((H2, 1), lambda i: (0, 0)),
            pl.BlockSpec((1, 1), lambda i: (0, 0)),
        ],
        out_specs=pl.BlockSpec((bt, 1), lambda i: (i, 0)),
        out_shape=jax.ShapeDtypeStruct((B, 1), jnp.float32),
    )


def kernel(user_data, movie_id, genres, genres_shape, user_table, movie_table,
           genre_table, W1, b1, W2, b2, Wfc, bfc):
    B = user_data.shape[0]
    E = user_table.shape[1]
    G = genres.shape[1]
    NG = genre_table.shape[0]
    H1 = W1.shape[1]
    H2 = W2.shape[1]

    u, m = _make_sc_gather(B, E)(user_data, movie_id, user_table, movie_table)

    mlp = _make_mlp(B, E, G, NG, H1, H2, bt=2048)
    out = mlp(u, m, genres, genres_shape.reshape(B, 1), genre_table,
              W1, b1.reshape(1, H1), W2, b2.reshape(1, H2),
              Wfc, bfc.reshape(1, 1))
    return out.squeeze(-1)
